# baseline (speedup 1.0000x reference)
import functools

import jax
import jax.numpy as jnp
from jax import lax
from jax.experimental import pallas as pl
from jax.experimental.pallas import tpu as pltpu
from jax.experimental.pallas import tpu_sc as plsc

HIDDEN = 1024
D_FF = 4096
NUM_EXPERTS = 16
CAPACITY_FACTOR = 1.0
MIN_CAPACITY = 4

_NC = 2
_NS = 16
_NW = _NC * _NS

_LANES = 16


def _sc_mesh():
  return plsc.VectorSubcoreMesh(
      core_axis_name="c", subcore_axis_name="s",
      num_cores=_NC, num_subcores=_NS)


def _cumsum0_exclusive(a):
  n = a.shape[0]
  out = a
  s = 1
  while s < n:
    shifted = jnp.concatenate(
        [jnp.zeros((s, a.shape[1]), out.dtype), out[:-s]], axis=0)
    out = out + shifted
    s *= 2
  return out - a


def _router_body(x_ref, wg_ref, s1_ref, s2_ref, g1_ref, g2_ref,
                 laux_ref, cnt_ref, xb16_ref, *, capacity):
  x = x_ref[...]
  wg = wg_ref[...]
  S, _ = x.shape
  E = wg.shape[1]
  logits = jnp.dot(x, wg, preferred_element_type=jnp.float32)

  m = jnp.max(logits, axis=1, keepdims=True)
  ex = jnp.exp(logits - m)
  gates = ex / jnp.sum(ex, axis=1, keepdims=True)

  lane = lax.broadcasted_iota(jnp.int32, (S, E), 1)
  idx1 = jnp.min(jnp.where(logits == m, lane, E), axis=1)
  mask1 = (lane == idx1[:, None]).astype(jnp.int32)
  neg = jnp.where(mask1 == 1, -jnp.inf, logits)
  m2 = jnp.max(neg, axis=1, keepdims=True)
  idx2 = jnp.min(jnp.where(neg == m2, lane, E), axis=1)
  mask2 = (lane == idx2[:, None]).astype(jnp.int32)

  loc1 = _cumsum0_exclusive(mask1)
  counts1 = jnp.sum(mask1, axis=0, keepdims=True)
  loc2 = _cumsum0_exclusive(mask2) + counts1

  me = jnp.mean(gates, axis=0)
  ce = jnp.mean(mask1.astype(jnp.float32), axis=0)
  laux_ref[...] = jnp.reshape(jnp.mean(me * ce) * (E * E), (1, 1))
  cnt_ref[...] = counts1

  mask1k = mask1 * (loc1 < capacity).astype(jnp.int32)
  mask2k = mask2 * (loc2 < capacity).astype(jnp.int32)
  loc1_s = jnp.sum(loc1 * mask1k, axis=1)
  loc2_s = jnp.sum(loc2 * mask2k, axis=1)
  m1f = mask1k.astype(jnp.float32)
  m2f = mask2k.astype(jnp.float32)
  gates1_s = jnp.sum(gates * m1f, axis=1)
  gates2_s = jnp.sum(gates * m2f, axis=1)
  denom = jnp.maximum(gates1_s + gates2_s, jnp.finfo(jnp.float32).eps)
  g1 = gates1_s / denom
  g2 = gates2_s / denom
  ones_l = jnp.ones((1, 128), jnp.float32)
  g1_ref[...] = g1[:, None] * ones_l
  g2_ref[...] = g2[:, None] * ones_l

  kept1 = jnp.sum(mask1k, axis=1) > 0
  kept2 = jnp.sum(mask2k, axis=1) > 0
  slot1 = idx1 * capacity + loc1_s
  slot2 = idx2 * capacity + loc2_s
  zero_row = E * capacity
  s1_ref[...] = jnp.where(kept1, slot1, zero_row)
  s2_ref[...] = jnp.where(kept2, slot2, zero_row)
  xb16_ref[...] = x.astype(jnp.bfloat16)


def _router_call(x, wg, capacity):
  S = x.shape[0]
  E = wg.shape[1]
  return pl.pallas_call(
      functools.partial(_router_body, capacity=capacity),
      out_shape=(
          jax.ShapeDtypeStruct((S,), jnp.int32),
          jax.ShapeDtypeStruct((S,), jnp.int32),
          jax.ShapeDtypeStruct((S, 128), jnp.float32),
          jax.ShapeDtypeStruct((S, 128), jnp.float32),
          jax.ShapeDtypeStruct((1, 1), jnp.float32),
          jax.ShapeDtypeStruct((1, E), jnp.int32),
          jax.ShapeDtypeStruct(x.shape, jnp.bfloat16),
      ),
  )(x, wg)


def _dispatch_body(x_hbm, s1_hbm, s2_hbm, g1_hbm, g2_hbm,
                   disp_hbm, gslot_hbm,
                   xb, gb1, gb2, i1, i2, isem, ssem, sem, gsem,
                   *, tokens_per_worker, chunk):
  wid = lax.axis_index("s") * _NC + lax.axis_index("c")
  nk = tokens_per_worker // chunk
  ci1 = pltpu.async_copy(s1_hbm.at[wid], i1, isem)
  ci2 = pltpu.async_copy(s2_hbm.at[wid], i2, isem)
  ci1.wait()
  ci2.wait()
  scatters = []
  for k in range(nk):
    base = wid * tokens_per_worker + k * chunk
    b = k % 3
    if len(scatters) >= 2:
      for c in scatters.pop(0):
        c.wait()
    st = (
        pltpu.async_copy(x_hbm.at[pl.ds(base, chunk)], xb.at[b], ssem),
        pltpu.async_copy(g1_hbm.at[pl.ds(base, chunk)], gb1.at[b], ssem),
        pltpu.async_copy(g2_hbm.at[pl.ds(base, chunk)], gb2.at[b], ssem),
    )
    for c in st:
      c.wait()
    scatters.append((
        pltpu.async_copy(xb.at[b], disp_hbm.at[i1.at[k]], sem),
        pltpu.async_copy(xb.at[b], disp_hbm.at[i2.at[k]], sem),
        pltpu.async_copy(gb1.at[b], gslot_hbm.at[i1.at[k]], gsem),
        pltpu.async_copy(gb2.at[b], gslot_hbm.at[i2.at[k]], gsem),
    ))
  for cps in scatters:
    for c in cps:
      c.wait()


def _dispatch_call(x, s1r, s2r, g1, g2, nslot_pad):
  S, D = x.shape
  GW = g1.shape[1]
  tokens_per_worker = S // _NW
  chunk = min(tokens_per_worker, 32)
  nk = tokens_per_worker // chunk
  return pl.kernel(
      functools.partial(_dispatch_body,
                        tokens_per_worker=tokens_per_worker, chunk=chunk),
      mesh=_sc_mesh(),
      out_type=(
          jax.ShapeDtypeStruct((nslot_pad, D), jnp.int32),
          jax.ShapeDtypeStruct((nslot_pad, GW), jnp.float32),
      ),
      scratch_types=[
          pltpu.VMEM((3, chunk, D), jnp.int32),
          pltpu.VMEM((3, chunk, GW), jnp.float32),
          pltpu.VMEM((3, chunk, GW), jnp.float32),
          pltpu.VMEM((nk, chunk), jnp.int32),
          pltpu.VMEM((nk, chunk), jnp.int32),
          pltpu.SemaphoreType.DMA,
          pltpu.SemaphoreType.DMA,
          pltpu.SemaphoreType.DMA,
          pltpu.SemaphoreType.DMA,
      ],
  )(x, s1r, s2r, g1, g2)


def _ffn_body(disp_ref, w1_ref, b1_ref, w2_ref, b2_ref, gs_ref, out_ref,
              *, ne, nf):
  e = pl.program_id(0)
  f = pl.program_id(1)

  @pl.when(e == ne)
  def _zero_block():
    out_ref[...] = jnp.zeros_like(out_ref)

  @pl.when(e < ne)
  def _compute():
    @pl.when(f == 0)
    def _init():
      out_ref[...] = jnp.zeros_like(out_ref)

    xb = disp_ref[...]
    h = jnp.dot(xb, w1_ref[0].astype(jnp.bfloat16),
                preferred_element_type=jnp.float32)
    h = jnp.maximum(h + b1_ref[0, 0], 0.0).astype(jnp.bfloat16)
    out_ref[...] += jnp.dot(h, w2_ref[0].astype(jnp.bfloat16),
                            preferred_element_type=jnp.float32)

    @pl.when(f == nf - 1)
    def _fini():
      out_ref[...] = (out_ref[...] + b2_ref[0]) * gs_ref[:, 0:1]


def _ffn_call(disp, gslot, w1, b1, w2, b2, capacity):
  E, D, F = w1.shape
  ft = 2048
  nf = F // ft
  b1r = b1.reshape(E, nf, 1, ft)
  b2r = b2.reshape(E, 1, D)
  nslot_pad = disp.shape[0]
  return pl.pallas_call(
      functools.partial(_ffn_body, ne=E, nf=nf),
      grid=(E + 1, nf),
      in_specs=[
          pl.BlockSpec((capacity, D), lambda e, f, _m=E - 1: (jnp.minimum(e, _m), 0)),
          pl.BlockSpec((1, D, ft), lambda e, f, _m=E - 1: (jnp.minimum(e, _m), 0, f)),
          pl.BlockSpec((1, nf, 1, ft), lambda e, f, _m=E - 1: (jnp.minimum(e, _m), 0, 0, 0)),
          pl.BlockSpec((1, ft, D), lambda e, f, _m=E - 1: (jnp.minimum(e, _m), f, 0)),
          pl.BlockSpec((1, 1, D), lambda e, f, _m=E - 1: (jnp.minimum(e, _m), 0, 0)),
          pl.BlockSpec((capacity, 128), lambda e, f: (e, 0)),
      ],
      out_specs=pl.BlockSpec((capacity, D), lambda e, f: (e, 0)),
      out_shape=jax.ShapeDtypeStruct((nslot_pad, D), jnp.float32),
  )(disp, w1, b1r, w2, b2r, gslot)


def _combine_body(eo_hbm, s1_hbm, s2_hbm, out_hbm, i1, i2, r1, r2, sem, wsem,
                  *, tokens_per_worker, chunk, d):
  wid = lax.axis_index("s") * _NC + lax.axis_index("c")
  nk = tokens_per_worker // chunk
  nlane = d // _LANES

  ci1 = pltpu.async_copy(s1_hbm.at[wid], i1, sem)
  ci2 = pltpu.async_copy(s2_hbm.at[wid], i2, sem)
  ci1.wait()
  ci2.wait()

  def _fetch(k):
    b = k % 3
    cp1 = pltpu.async_copy(eo_hbm.at[i1.at[k]], r1.at[b], sem)
    cp2 = pltpu.async_copy(eo_hbm.at[i2.at[k]], r2.at[b], sem)
    return cp1, cp2

  pending = _fetch(0)
  writes = []
  for k in range(nk):
    b = k % 3
    base = wid * tokens_per_worker + k * chunk
    pending[0].wait()
    pending[1].wait()
    if k + 1 < nk:
      if len(writes) >= 2:
        writes.pop(0).wait()
      pending = _fetch(k + 1)
    for t in range(chunk):
      def _add(ci, t=t, b=b):
        off = ci * _LANES
        r1[b, t, pl.ds(off, _LANES)] = (
            r1[b, t, pl.ds(off, _LANES)] + r2[b, t, pl.ds(off, _LANES)])
      plsc.parallel_loop(0, nlane, unroll=8)(_add)
    writes.append(
        pltpu.async_copy(r1.at[b], out_hbm.at[pl.ds(base, chunk)], wsem))
  for w in writes:
    w.wait()


def _combine_call(eo, s1, s2, S):
  D = eo.shape[1]
  tokens_per_worker = S // _NW
  chunk = min(tokens_per_worker, 16)
  return pl.kernel(
      functools.partial(_combine_body,
                        tokens_per_worker=tokens_per_worker,
                        chunk=chunk, d=D),
      mesh=_sc_mesh(),
      out_type=jax.ShapeDtypeStruct((S, D), jnp.float32),
      scratch_types=[
          pltpu.VMEM((tokens_per_worker // chunk, chunk), jnp.int32),
          pltpu.VMEM((tokens_per_worker // chunk, chunk), jnp.int32),
          pltpu.VMEM((3, chunk, D), jnp.float32),
          pltpu.VMEM((3, chunk, D), jnp.float32),
          pltpu.SemaphoreType.DMA,
          pltpu.SemaphoreType.DMA,
      ],
  )(eo, s1, s2)


def kernel(hidden_states, wg, w1, b1, w2, b2):
  B, T, D = hidden_states.shape
  S = B * T
  E = wg.shape[1]
  capacity = max(int(2 * S / E * CAPACITY_FACTOR), MIN_CAPACITY)
  nslot_pad = E * capacity + capacity

  x = hidden_states.reshape(S, D)
  s1, s2, g1, g2, laux, cnt, xb16 = _router_call(x, wg, capacity)
  xi32 = lax.bitcast_convert_type(xb16.reshape(S, D // 2, 2), jnp.int32)
  tpw = S // _NW
  s1d = s1.reshape(_NW, tpw // 32, 32)
  s2d = s2.reshape(_NW, tpw // 32, 32)
  s1c = s1.reshape(_NW, tpw // 16, 16)
  s2c = s2.reshape(_NW, tpw // 16, 16)
  disp_i, gslot = _dispatch_call(xi32, s1d, s2d, g1, g2, nslot_pad)
  disp = lax.bitcast_convert_type(disp_i, jnp.bfloat16).reshape(nslot_pad, D)
  eo = _ffn_call(disp, gslot, w1, b1, w2, b2, capacity)
  out = _combine_call(eo, s1c, s2c, S)
  return out.reshape(B, T, D), laux.reshape(()), cnt.reshape(E)


# speedup vs baseline: 1.8455x; 1.8455x over previous
import functools

import jax
import jax.numpy as jnp
from jax import lax
from jax.experimental import pallas as pl
from jax.experimental.pallas import tpu as pltpu
from jax.experimental.pallas import tpu_sc as plsc

HIDDEN = 1024
D_FF = 4096
NUM_EXPERTS = 16
CAPACITY_FACTOR = 1.0
MIN_CAPACITY = 4

_NC = 2
_NS = 16
_NW = _NC * _NS

_LANES = 16


def _sc_mesh():
  return plsc.VectorSubcoreMesh(
      core_axis_name="c", subcore_axis_name="s",
      num_cores=_NC, num_subcores=_NS)


def _cumsum0_exclusive(a):
  n = a.shape[0]
  out = a
  s = 1
  while s < n:
    shifted = jnp.concatenate(
        [jnp.zeros((s, a.shape[1]), out.dtype), out[:-s]], axis=0)
    out = out + shifted
    s *= 2
  return out - a


def _router_body(x_ref, wg_ref, s1_ref, s2_ref, g1_ref, g2_ref,
                 laux_ref, cnt_ref, *, capacity):
  x = x_ref[...]
  wg = wg_ref[...]
  S, _ = x.shape
  E = wg.shape[1]
  logits = jnp.dot(x, wg, preferred_element_type=jnp.float32)

  m = jnp.max(logits, axis=1, keepdims=True)
  ex = jnp.exp(logits - m)
  gates = ex / jnp.sum(ex, axis=1, keepdims=True)

  lane = lax.broadcasted_iota(jnp.int32, (S, E), 1)
  idx1 = jnp.min(jnp.where(logits == m, lane, E), axis=1)
  mask1 = (lane == idx1[:, None]).astype(jnp.int32)
  neg = jnp.where(mask1 == 1, -jnp.inf, logits)
  m2 = jnp.max(neg, axis=1, keepdims=True)
  idx2 = jnp.min(jnp.where(neg == m2, lane, E), axis=1)
  mask2 = (lane == idx2[:, None]).astype(jnp.int32)

  loc1 = _cumsum0_exclusive(mask1)
  counts1 = jnp.sum(mask1, axis=0, keepdims=True)
  loc2 = _cumsum0_exclusive(mask2) + counts1

  me = jnp.mean(gates, axis=0)
  ce = jnp.mean(mask1.astype(jnp.float32), axis=0)
  laux_ref[...] = jnp.reshape(jnp.mean(me * ce) * (E * E), (1, 1))
  cnt_ref[...] = counts1

  mask1k = mask1 * (loc1 < capacity).astype(jnp.int32)
  mask2k = mask2 * (loc2 < capacity).astype(jnp.int32)
  loc1_s = jnp.sum(loc1 * mask1k, axis=1)
  loc2_s = jnp.sum(loc2 * mask2k, axis=1)
  m1f = mask1k.astype(jnp.float32)
  m2f = mask2k.astype(jnp.float32)
  gates1_s = jnp.sum(gates * m1f, axis=1)
  gates2_s = jnp.sum(gates * m2f, axis=1)
  denom = jnp.maximum(gates1_s + gates2_s, jnp.finfo(jnp.float32).eps)
  g1 = gates1_s / denom
  g2 = gates2_s / denom
  ones_l = jnp.ones((1, 128), jnp.float32)
  g1_ref[...] = g1[:, None] * ones_l
  g2_ref[...] = g2[:, None] * ones_l

  kept1 = jnp.sum(mask1k, axis=1) > 0
  kept2 = jnp.sum(mask2k, axis=1) > 0
  slot1 = idx1 * capacity + loc1_s
  slot2 = idx2 * capacity + loc2_s
  zero_row = E * capacity
  s1_ref[...] = jnp.where(kept1, slot1, zero_row)
  s2_ref[...] = jnp.where(kept2, slot2, zero_row)


def _router_call(x, wg, capacity):
  S = x.shape[0]
  E = wg.shape[1]
  return pl.pallas_call(
      functools.partial(_router_body, capacity=capacity),
      out_shape=(
          jax.ShapeDtypeStruct((S,), jnp.int32),
          jax.ShapeDtypeStruct((S,), jnp.int32),
          jax.ShapeDtypeStruct((S, 128), jnp.float32),
          jax.ShapeDtypeStruct((S, 128), jnp.float32),
          jax.ShapeDtypeStruct((1, 1), jnp.float32),
          jax.ShapeDtypeStruct((1, E), jnp.int32),
      ),
  )(x, wg)


def _dispatch_body(x_hbm, s1_hbm, s2_hbm, g1_hbm, g2_hbm,
                   disp_hbm, gslot_hbm,
                   xb, gb1, gb2, i1, i2, isem, ssem, sem, gsem,
                   *, tokens_per_worker, chunk):
  wid = lax.axis_index("s") * _NC + lax.axis_index("c")
  nk = tokens_per_worker // chunk
  tbase = wid * tokens_per_worker
  ci1 = pltpu.async_copy(s1_hbm.at[wid], i1, isem)
  ci2 = pltpu.async_copy(s2_hbm.at[wid], i2, isem)
  cg1 = pltpu.async_copy(g1_hbm.at[pl.ds(tbase, tokens_per_worker)], gb1, isem)
  cg2 = pltpu.async_copy(g2_hbm.at[pl.ds(tbase, tokens_per_worker)], gb2, isem)
  ci1.wait()
  ci2.wait()
  cg1.wait()
  cg2.wait()
  scatters = []
  for k in range(nk):
    b = k % 2
    if len(scatters) >= 2:
      for c in scatters.pop(0):
        c.wait()
    pltpu.async_copy(
        x_hbm.at[pl.ds(tbase + k * chunk, chunk)], xb.at[b], ssem).wait()
    scatters.append((
        pltpu.async_copy(xb.at[b], disp_hbm.at[i1.at[k]], sem),
        pltpu.async_copy(xb.at[b], disp_hbm.at[i2.at[k]], sem),
        pltpu.async_copy(gb1.at[pl.ds(k * chunk, chunk)],
                         gslot_hbm.at[i1.at[k]], gsem),
        pltpu.async_copy(gb2.at[pl.ds(k * chunk, chunk)],
                         gslot_hbm.at[i2.at[k]], gsem),
    ))
  for cps in scatters:
    for c in cps:
      c.wait()


def _dispatch_call(x, s1r, s2r, g1, g2, nslot_pad):
  S, D = x.shape
  GW = g1.shape[1]
  tokens_per_worker = S // _NW
  chunk = min(tokens_per_worker, 32)
  nk = tokens_per_worker // chunk
  return pl.kernel(
      functools.partial(_dispatch_body,
                        tokens_per_worker=tokens_per_worker, chunk=chunk),
      mesh=_sc_mesh(),
      out_type=(
          jax.ShapeDtypeStruct((nslot_pad, D), jnp.float32),
          jax.ShapeDtypeStruct((nslot_pad, GW), jnp.float32),
      ),
      scratch_types=[
          pltpu.VMEM((2, chunk, D), jnp.float32),
          pltpu.VMEM((tokens_per_worker, GW), jnp.float32),
          pltpu.VMEM((tokens_per_worker, GW), jnp.float32),
          pltpu.VMEM((nk, chunk), jnp.int32),
          pltpu.VMEM((nk, chunk), jnp.int32),
          pltpu.SemaphoreType.DMA,
          pltpu.SemaphoreType.DMA,
          pltpu.SemaphoreType.DMA,
          pltpu.SemaphoreType.DMA,
      ],
  )(x, s1r, s2r, g1, g2)


def _ffn_body(disp_ref, w1_ref, b1_ref, w2_ref, b2_ref, gs_ref, out_ref,
              *, ne, nf):
  e = pl.program_id(0)
  f = pl.program_id(1)

  @pl.when(e == ne)
  def _zero_block():
    out_ref[...] = jnp.zeros_like(out_ref)

  @pl.when(e < ne)
  def _compute():
    @pl.when(f == 0)
    def _init():
      out_ref[...] = jnp.zeros_like(out_ref)

    xb = disp_ref[...].astype(jnp.bfloat16)
    h = jnp.dot(xb, w1_ref[0].astype(jnp.bfloat16),
                preferred_element_type=jnp.float32)
    h = jnp.maximum(h + b1_ref[0, 0], 0.0).astype(jnp.bfloat16)
    out_ref[...] += jnp.dot(h, w2_ref[0].astype(jnp.bfloat16),
                            preferred_element_type=jnp.float32)

    @pl.when(f == nf - 1)
    def _fini():
      out_ref[...] = (out_ref[...] + b2_ref[0]) * gs_ref[:, 0:1]


def _ffn_call(disp, gslot, w1, b1, w2, b2, capacity):
  E, D, F = w1.shape
  ft = 2048
  nf = F // ft
  b1r = b1.reshape(E, nf, 1, ft)
  b2r = b2.reshape(E, 1, D)
  nslot_pad = disp.shape[0]
  return pl.pallas_call(
      functools.partial(_ffn_body, ne=E, nf=nf),
      grid=(E + 1, nf),
      in_specs=[
          pl.BlockSpec((capacity, D), lambda e, f, _m=E - 1: (jnp.minimum(e, _m), 0)),
          pl.BlockSpec((1, D, ft), lambda e, f, _m=E - 1: (jnp.minimum(e, _m), 0, f)),
          pl.BlockSpec((1, nf, 1, ft), lambda e, f, _m=E - 1: (jnp.minimum(e, _m), 0, 0, 0)),
          pl.BlockSpec((1, ft, D), lambda e, f, _m=E - 1: (jnp.minimum(e, _m), f, 0)),
          pl.BlockSpec((1, 1, D), lambda e, f, _m=E - 1: (jnp.minimum(e, _m), 0, 0)),
          pl.BlockSpec((capacity, 128), lambda e, f: (e, 0)),
      ],
      out_specs=pl.BlockSpec((capacity, D), lambda e, f: (e, 0)),
      out_shape=jax.ShapeDtypeStruct((nslot_pad, D), jnp.float32),
  )(disp, w1, b1r, w2, b2r, gslot)


def _combine_body(eo_hbm, s12_hbm, out_hbm, i12, r12, ov, sem, wsem,
                  *, tokens_per_worker, chunk, d):
  wid = lax.axis_index("s") * _NC + lax.axis_index("c")
  nk = tokens_per_worker // chunk
  nlane = d // _LANES

  pltpu.async_copy(s12_hbm.at[wid], i12, sem).wait()

  def _fetch(k):
    return pltpu.async_copy(eo_hbm.at[i12.at[k]], r12.at[k % 2], sem)

  pending = _fetch(0)
  writes = []
  for k in range(nk):
    b = k % 2
    base = wid * tokens_per_worker + k * chunk
    pending.wait()
    if k + 1 < nk:
      pending = _fetch(k + 1)
    if len(writes) >= 2:
      writes.pop(0).wait()
    for t in range(chunk):
      def _add(ci, t=t, b=b):
        off = ci * _LANES
        ov[b, t, pl.ds(off, _LANES)] = (
            r12[b, t, pl.ds(off, _LANES)] + r12[b, t + chunk, pl.ds(off, _LANES)])
      plsc.parallel_loop(0, nlane, unroll=8)(_add)
    writes.append(
        pltpu.async_copy(ov.at[b], out_hbm.at[pl.ds(base, chunk)], wsem))
  for w in writes:
    w.wait()


def _combine_call(eo, s12, S):
  D = eo.shape[1]
  tokens_per_worker = S // _NW
  chunk = min(tokens_per_worker, 16)
  nk = tokens_per_worker // chunk
  return pl.kernel(
      functools.partial(_combine_body,
                        tokens_per_worker=tokens_per_worker,
                        chunk=chunk, d=D),
      mesh=_sc_mesh(),
      out_type=jax.ShapeDtypeStruct((S, D), jnp.float32),
      scratch_types=[
          pltpu.VMEM((nk, 2 * chunk), jnp.int32),
          pltpu.VMEM((2, 2 * chunk, D), jnp.float32),
          pltpu.VMEM((2, chunk, D), jnp.float32),
          pltpu.SemaphoreType.DMA,
          pltpu.SemaphoreType.DMA,
      ],
  )(eo, s12)


def kernel(hidden_states, wg, w1, b1, w2, b2):
  B, T, D = hidden_states.shape
  S = B * T
  E = wg.shape[1]
  capacity = max(int(2 * S / E * CAPACITY_FACTOR), MIN_CAPACITY)
  nslot_pad = E * capacity + capacity

  x = hidden_states.reshape(S, D)
  s1, s2, g1, g2, laux, cnt = _router_call(x, wg, capacity)
  tpw = S // _NW
  s1d = s1.reshape(_NW, tpw // 32, 32)
  s2d = s2.reshape(_NW, tpw // 32, 32)
  s12 = jnp.concatenate(
      [s1.reshape(_NW, tpw // 16, 16), s2.reshape(_NW, tpw // 16, 16)],
      axis=2)
  disp, gslot = _dispatch_call(x, s1d, s2d, g1, g2, nslot_pad)
  eo = _ffn_call(disp, gslot, w1, b1, w2, b2, capacity)
  out = _combine_call(eo, s12, S)
  return out.reshape(B, T, D), laux.reshape(()), cnt.reshape(E)
